# baseline (speedup 1.0000x reference)
import functools
import math

import jax
import jax.numpy as jnp
from jax import lax
from jax.experimental import pallas as pl
from jax.experimental.pallas import tpu as pltpu
from jax.experimental.pallas import tpu_sc as plsc

VOCAB = 100000
D = 768
B = 4
S = 4096
N = B * S
SCALE = math.sqrt(float(D))

_info = plsc.get_sparse_core_info()
NC = _info.num_cores
NS = _info.num_subcores
NW = NC * NS
S_W = S // NW
R = 32
NCH = S_W // R
LANES = 16
JV = D // LANES


def _sc_embed(idx_arr, table, pe_s):
    mesh = plsc.VectorSubcoreMesh(core_axis_name="c", subcore_axis_name="s")

    @functools.partial(
        pl.kernel,
        mesh=mesh,
        out_type=jax.ShapeDtypeStruct((N, D), jnp.float32),
        scratch_types=[
            pltpu.VMEM((NCH * B, R), jnp.int32),
            pltpu.VMEM((B, R, D), jnp.float32),
            pltpu.VMEM((R, D), jnp.float32),
            pltpu.SemaphoreType.DMA((B,)),
            pltpu.SemaphoreType.DMA((B,)),
        ],
    )
    def k(idx_hbm, table_hbm, pe_hbm, out_hbm,
          idx_v, rows_v, pe_v, g_sem, o_sem):
        wid = lax.axis_index("s") * NC + lax.axis_index("c")
        sbase = wid * S_W

        def fire_gather(u, buf):
            pltpu.async_copy(
                table_hbm.at[idx_v.at[u]], rows_v.at[buf], g_sem.at[buf])

        def drain_gather(u, buf):
            pltpu.make_async_copy(
                table_hbm.at[idx_v.at[u]], rows_v.at[buf],
                g_sem.at[buf]).wait()

        def drain_out(buf):
            pltpu.make_async_copy(
                rows_v.at[buf], out_hbm.at[pl.ds(0, R)], o_sem.at[buf]).wait()

        pltpu.sync_copy(idx_hbm.at[wid], idx_v)
        fire_gather(0, 0)

        def chunk(sc, _):
            pltpu.sync_copy(pe_hbm.at[pl.ds(sbase + sc * R, R)], pe_v)
            for b in range(B):
                u = sc * B + b
                nbuf = (b + 1) % B
                if b == B - 1:
                    def pf():
                        drain_out(nbuf)
                        fire_gather(u + 1, nbuf)
                    pl.when(sc < NCH - 1)(pf)
                else:
                    pl.when(sc >= 1)(lambda: drain_out(nbuf))
                    fire_gather(u + 1, nbuf)
                drain_gather(u, b)

                def row(r, _):
                    for j in range(JV):
                        sl = pl.ds(j * LANES, LANES)
                        rows_v[b, r, sl] = (
                            rows_v[b, r, sl] * SCALE + pe_v[r, sl])
                    return 0

                lax.fori_loop(0, R, row, 0)
                pltpu.async_copy(
                    rows_v.at[b],
                    out_hbm.at[pl.ds(b * S + sbase + sc * R, R)],
                    o_sem.at[b])
            return 0

        lax.fori_loop(0, NCH, chunk, 0)
        for buf in range(B):
            drain_out(buf)

    return k(idx_arr, table, pe_s)


def kernel(x, table, pe):
    idx_arr = (x.reshape(B, NW, NCH, R)
                .transpose(1, 2, 0, 3)
                .reshape(NW, NCH * B, R))
    out = _sc_embed(idx_arr, table, pe[:S])
    return out.reshape(B, S, D)


# speedup vs baseline: 1.0373x; 1.0373x over previous
import functools
import math

import jax
import jax.numpy as jnp
from jax import lax
from jax.experimental import pallas as pl
from jax.experimental.pallas import tpu as pltpu
from jax.experimental.pallas import tpu_sc as plsc

VOCAB = 100000
D = 768
B = 4
S = 4096
N = B * S
SCALE = math.sqrt(float(D))

_info = plsc.get_sparse_core_info()
NC = _info.num_cores
NS = _info.num_subcores
NW = NC * NS
S_W = S // NW
R = 32
NCH = S_W // R
NU = NCH * B
LANES = 16
JV = D // LANES


def _sc_embed(idx_arr, table, pe_s):
    mesh = plsc.VectorSubcoreMesh(core_axis_name="c", subcore_axis_name="s")

    @functools.partial(
        pl.kernel,
        mesh=mesh,
        out_type=jax.ShapeDtypeStruct((N, D), jnp.float32),
        scratch_types=[
            pltpu.VMEM((NU, R), jnp.int32),
            pltpu.VMEM((3, R, D), jnp.float32),
            pltpu.VMEM((2, R, D), jnp.float32),
            pltpu.SemaphoreType.DMA((3,)),
            pltpu.SemaphoreType.DMA((3,)),
            pltpu.SemaphoreType.DMA((2,)),
        ],
    )
    def k(idx_hbm, table_hbm, pe_hbm, out_hbm,
          idx_v, rows_v, pe_v, g_sem, o_sem, p_sem):
        wid = lax.axis_index("s") * NC + lax.axis_index("c")
        sbase = wid * S_W

        pltpu.sync_copy(idx_hbm.at[wid], idx_v)
        pe_h = [None, None]
        pe_h[0] = pltpu.async_copy(
            pe_hbm.at[pl.ds(sbase, R)], pe_v.at[0], p_sem.at[0])
        g_h = [None] * 3
        o_h = [None] * 3
        g_h[0] = pltpu.async_copy(
            table_hbm.at[idx_v.at[0]], rows_v.at[0], g_sem.at[0])

        for u in range(NU):
            sc, b = divmod(u, B)
            par = sc % 2
            if b == 0:
                if sc + 1 < NCH:
                    pe_h[1 - par] = pltpu.async_copy(
                        pe_hbm.at[pl.ds(sbase + (sc + 1) * R, R)],
                        pe_v.at[1 - par], p_sem.at[1 - par])
                pe_h[par].wait()
            buf = u % 3
            if u + 1 < NU:
                nbuf = (u + 1) % 3
                if o_h[nbuf] is not None:
                    o_h[nbuf].wait()
                g_h[nbuf] = pltpu.async_copy(
                    table_hbm.at[idx_v.at[u + 1]], rows_v.at[nbuf],
                    g_sem.at[nbuf])
            g_h[buf].wait()

            def row(i, _, buf=buf, par=par):
                r = i // 2
                h = (i % 2) * (JV // 2 * LANES)
                for j in range(JV // 2):
                    sl = pl.ds(h + j * LANES, LANES)
                    rows_v[buf, r, sl] = (
                        rows_v[buf, r, sl] * SCALE + pe_v[par, r, sl])
                return 0

            lax.fori_loop(0, 2 * R, row, 0)
            o_h[buf] = pltpu.async_copy(
                rows_v.at[buf],
                out_hbm.at[pl.ds(b * S + sbase + sc * R, R)], o_sem.at[buf])

        for buf in range(3):
            if o_h[buf] is not None:
                o_h[buf].wait()

    return k(idx_arr, table, pe_s)


def kernel(x, table, pe):
    idx_arr = (x.reshape(B, NW, NCH, R)
                .transpose(1, 2, 0, 3)
                .reshape(NW, NCH * B, R))
    out = _sc_embed(idx_arr, table, pe[:S])
    return out.reshape(B, S, D)
